# Initial kernel scaffold; baseline (speedup 1.0000x reference)
import jax
import jax.numpy as jnp
from jax.experimental import pallas as pl


def kernel(x, edge_index, timestamps, params):
    raise NotImplementedError("write your pallas kernel here")


# speedup vs baseline: 17.8887x; 17.8887x over previous
import functools

import jax
import jax.numpy as jnp
from jax import lax
from jax.experimental import pallas as pl
from jax.experimental.pallas import tpu as pltpu
from jax.experimental.pallas import tpu_sc as plsc


def _median_body(tb_ref, out_ref, *, kp1):
    tb = tb_ref[...]

    def body(_, lohi):
        lo, hi = lohi
        mid = (lo + hi) // 2
        cnt = jnp.sum((tb <= mid).astype(jnp.int32))
        ok = cnt >= kp1
        return jnp.where(ok, lo, mid + 1), jnp.where(ok, mid, hi)

    _, hi = lax.fori_loop(0, 31, body, (jnp.int32(0), jnp.int32(2**31 - 1)))
    out_ref[...] = jnp.full((8, 128), hi, jnp.int32)


def _median_bits(tbits2d, kp1):
    return pl.pallas_call(
        functools.partial(_median_body, kp1=kp1),
        out_shape=jax.ShapeDtypeStruct((8, 128), jnp.int32),
    )(tbits2d)


def _make_sc1(E, NP, DUMP, NC, NS):
    EW = E // (NC * NS)
    NB = EW // 80
    RPW = NP // NS
    mesh = plsc.VectorSubcoreMesh(core_axis_name="c", subcore_axis_name="s")

    @functools.partial(
        pl.kernel,
        out_type=jax.ShapeDtypeStruct((8, NP), jnp.float32),
        mesh=mesh,
        scratch_types=[
            pltpu.VMEM((EW,), jnp.int32),
            pltpu.VMEM((EW,), jnp.float32),
            pltpu.VMEM((NP,), jnp.float32),
            pltpu.VMEM((80,), jnp.int32),
            pltpu.VMEM((80,), jnp.int32),
            pltpu.VMEM((80,), jnp.float32),
            pltpu.VMEM((RPW,), jnp.float32),
            pltpu.VMEM((RPW,), jnp.float32),
            pltpu.VMEM((16,), jnp.float32),
            pltpu.VMEM_SHARED((NS, NP), jnp.float32),
            pltpu.VMEM_SHARED((NP,), jnp.float32),
            pltpu.VMEM_SHARED((NP,), jnp.float32),
        ],
    )
    def sc1(dst_hbm, ts_hbm, med_hbm, out_hbm, dstv, tsv, ntb, idxr, idxh,
            ones80, red, acc, medv, snt, sdegr, sdegh):
        c = lax.axis_index("c")
        s = lax.axis_index("s")
        gid = c * NS + s

        pltpu.sync_copy(med_hbm, medv)
        mvec = medv[...]

        def init_nt(i, _):
            ntb[pl.ds(i * 16, 16)] = jnp.full((16,), -jnp.inf, jnp.float32)
            return 0
        lax.fori_loop(0, NP // 16, init_nt, 0)

        def init_acc(i, _):
            acc[pl.ds(i * 16, 16)] = jnp.zeros((16,), jnp.float32)
            return 0
        lax.fori_loop(0, RPW // 16, init_acc, 0)
        for j in range(5):
            ones80[pl.ds(j * 16, 16)] = jnp.ones((16,), jnp.float32)

        pltpu.sync_copy(acc, sdegr.at[pl.ds(s * RPW, RPW)])
        pltpu.sync_copy(acc, sdegh.at[pl.ds(s * RPW, RPW)])
        plsc.subcore_barrier()

        base = gid * EW
        pltpu.sync_copy(dst_hbm.at[pl.ds(base, EW)], dstv)
        pltpu.sync_copy(ts_hbm.at[pl.ds(base, EW)], tsv)

        i16 = lax.iota(jnp.int32, 16)

        def batch(b, _):
            for j in range(5):
                off = b * 80 + j * 16
                d = dstv[pl.ds(off, 16)]
                t = tsv[pl.ds(off, 16)]
                rec = t >= mvec
                idxr[pl.ds(j * 16, 16)] = jnp.where(rec, d, DUMP)
                idxh[pl.ds(j * 16, 16)] = jnp.where(rec, DUMP, d)

                for k in range(16):
                    dk = d[k]
                    tk = t[k]
                    boff = (dk // 16) * 16
                    lanei = dk - boff
                    cur = ntb[pl.ds(boff, 16)]
                    ntb[pl.ds(boff, 16)] = jnp.where(
                        i16 == lanei, jnp.maximum(cur, tk), cur)
            pltpu.sync_copy(ones80, sdegr.at[idxr], add=True)
            pltpu.sync_copy(ones80, sdegh.at[idxh], add=True)
            return 0

        lax.fori_loop(0, NB, batch, 0)

        pltpu.sync_copy(ntb, snt.at[s])
        plsc.subcore_barrier()

        def init_r(i, _):
            acc[pl.ds(i * 16, 16)] = jnp.full((16,), -jnp.inf, jnp.float32)
            return 0
        lax.fori_loop(0, RPW // 16, init_r, 0)

        def red_k(k2, _):
            pltpu.sync_copy(snt.at[k2, pl.ds(s * RPW, RPW)], red)

            def mx(i, _):
                acc[pl.ds(i * 16, 16)] = jnp.maximum(
                    acc[pl.ds(i * 16, 16)], red[pl.ds(i * 16, 16)])
                return 0

            lax.fori_loop(0, RPW // 16, mx, 0)
            return 0

        lax.fori_loop(0, NS, red_k, 0)

        pltpu.sync_copy(acc, out_hbm.at[c, pl.ds(s * RPW, RPW)])
        pltpu.sync_copy(sdegr.at[pl.ds(s * RPW, RPW)],
                        out_hbm.at[2 + c, pl.ds(s * RPW, RPW)])
        pltpu.sync_copy(sdegh.at[pl.ds(s * RPW, RPW)],
                        out_hbm.at[4 + c, pl.ds(s * RPW, RPW)])

    return sc1


def _make_sc2(E, N, D, NP, DUMP, NC, NS):
    EW = E // NS
    NB = EW // 80
    RPW = NP // NS
    mesh = plsc.VectorSubcoreMesh(core_axis_name="c", subcore_axis_name="s")

    @functools.partial(
        pl.kernel,
        out_type=jax.ShapeDtypeStruct((2, NP, D), jnp.float32),
        mesh=mesh,
        scratch_types=[
            pltpu.VMEM((2000,), jnp.int32),
            pltpu.VMEM((2000,), jnp.int32),
            pltpu.VMEM((2000,), jnp.float32),
            pltpu.VMEM((80,), jnp.int32),
            pltpu.VMEM((80,), jnp.int32),
            pltpu.VMEM((80, D), jnp.float32),
            pltpu.VMEM((16,), jnp.float32),
            pltpu.VMEM_SHARED((NP, D), jnp.float32),
            pltpu.SemaphoreType.DMA,
        ],
    )
    def sc2(src_hbm, dst_hbm, ts_hbm, med_hbm, g_hbm, out_hbm, srcv, dstv,
            tsv, sidx, gidx, rows, medv, acc, gsem):
        c = lax.axis_index("c")
        s = lax.axis_index("s")

        pltpu.sync_copy(med_hbm, medv)
        mvec = medv[...]

        def zr(i, _):
            def zc(k, _):
                rows[i, pl.ds(k * 16, 16)] = jnp.zeros((16,), jnp.float32)
                return 0
            lax.fori_loop(0, D // 16, zc, 0)
            return 0
        lax.fori_loop(0, 80, zr, 0)

        def za(r, _):
            pltpu.sync_copy(rows, acc.at[pl.ds(s * RPW + r * 80, 80)])
            return 0
        lax.fori_loop(0, RPW // 80, za, 0)
        plsc.subcore_barrier()

        base = s * EW
        goff = c * N

        def superb(sb, _):
            sbase = base + sb * 2000
            pltpu.sync_copy(src_hbm.at[pl.ds(sbase, 2000)], srcv)
            pltpu.sync_copy(dst_hbm.at[pl.ds(sbase, 2000)], dstv)
            pltpu.sync_copy(ts_hbm.at[pl.ds(sbase, 2000)], tsv)

            def batch(b, _):
                for j in range(5):
                    off = b * 80 + j * 16
                    d = dstv[pl.ds(off, 16)]
                    t = tsv[pl.ds(off, 16)]
                    reci = jnp.where(t >= mvec, 1, 0)
                    mine = reci == (1 - c)
                    sidx[pl.ds(j * 16, 16)] = jnp.where(mine, d, DUMP)
                    gidx[pl.ds(j * 16, 16)] = srcv[pl.ds(off, 16)] + goff
                pltpu.async_copy(g_hbm.at[gidx], rows, gsem).wait()
                pltpu.sync_copy(rows, acc.at[sidx], add=True)
                return 0

            lax.fori_loop(0, 25, batch, 0)
            return 0

        lax.fori_loop(0, EW // 2000, superb, 0)
        plsc.subcore_barrier()

        def wout(r, _):
            pltpu.sync_copy(acc.at[pl.ds(s * RPW + r * 80, 80)],
                            out_hbm.at[c, pl.ds(s * RPW + r * 80, 80)])
            return 0

        lax.fori_loop(0, RPW // 80, wout, 0)

    return sc2


def _prescale_body(x_ref, pb_ref, pf_ref, win_ref, cst_ref, g_ref, dinv_ref):
    pf = pf_ref[...]
    ntc_f = jnp.maximum(pf[:, 0:1], pf[:, 1:2])
    ntf_f = jnp.where(ntc_f < -3e38, 0.0, ntc_f)
    mn = jnp.min(ntf_f)
    mx = jnp.max(ntf_f)

    pb = pb_ref[...]
    ntc = jnp.maximum(pb[:, 0:1], pb[:, 1:2])
    ntf = jnp.where(ntc < -3e38, 0.0, ntc)
    ntn = (ntf - mn) / (mx - mn + 1e-8)

    cst = cst_ref[...]
    h = jnp.dot(x_ref[...], win_ref[...].T,
                preferred_element_type=jnp.float32)
    h = jnp.maximum(h + cst[0:1, :] + ntn * cst[1:2, :] + cst[2:3, :], 0.0)
    dr = lax.rsqrt(pb[:, 2:3] + pb[:, 3:4] + 1.0)
    dh = lax.rsqrt(pb[:, 4:5] + pb[:, 5:6] + 1.0)
    g_ref[...] = jnp.stack([dr * h, dh * h])
    z = jnp.zeros_like(pb[:, 0:6])
    dinv_ref[...] = jnp.concatenate([dr, dh, z], axis=1)


def _prescale(x, parts_t, win, cst, BN):
    N, F = x.shape
    D = win.shape[0]
    grid = (N // BN,)
    return pl.pallas_call(
        _prescale_body,
        grid=grid,
        in_specs=[
            pl.BlockSpec((BN, F), lambda i: (i, 0)),
            pl.BlockSpec((BN, 8), lambda i: (i, 0)),
            pl.BlockSpec((N, 8), lambda i: (0, 0)),
            pl.BlockSpec((D, F), lambda i: (0, 0)),
            pl.BlockSpec((8, D), lambda i: (0, 0)),
        ],
        out_specs=[
            pl.BlockSpec((2, BN, D), lambda i: (0, i, 0)),
            pl.BlockSpec((BN, 8), lambda i: (i, 0)),
        ],
        out_shape=[
            jax.ShapeDtypeStruct((2, N, D), jnp.float32),
            jax.ShapeDtypeStruct((N, 8), jnp.float32),
        ],
    )(x, parts_t, parts_t, win, cst)


def _epilogue_body(a_ref, g_ref, dinv_ref, wr_ref, wh_ref, m_ref, wc1_ref,
                   wc2_ref, cst_ref, out_ref):
    cst = cst_ref[...]
    dv = dinv_ref[...]
    ar = (a_ref[0] + g_ref[0]) * dv[:, 0:1]
    ah = (a_ref[1] + g_ref[1]) * dv[:, 1:2]
    h_rec = jnp.dot(ar, wr_ref[...].T,
                    preferred_element_type=jnp.float32) + cst[0:1, :]
    h_his = jnp.dot(ah, wh_ref[...].T,
                    preferred_element_type=jnp.float32) + cst[1:2, :]
    m_r = jnp.mean(h_rec, axis=1, keepdims=True)
    m_h = jnp.mean(h_his, axis=1, keepdims=True)
    mxw = jnp.maximum(m_r, m_h)
    er = jnp.exp(m_r - mxw)
    eh = jnp.exp(m_h - mxw)
    den = er + eh
    summ = (er / den) * h_rec + (eh / den) * h_his

    mu = jnp.mean(summ, axis=1, keepdims=True)
    var = jnp.mean((summ - mu) ** 2, axis=1, keepdims=True)
    agg = (summ - mu) / jnp.sqrt(var + 1e-5) * cst[2:3, :] + cst[3:4, :]

    att = jnp.dot(agg, m_ref[...].T,
                  preferred_element_type=jnp.float32) + cst[4:5, :]
    z = agg + att
    mu2 = jnp.mean(z, axis=1, keepdims=True)
    var2 = jnp.mean((z - mu2) ** 2, axis=1, keepdims=True)
    h2 = (z - mu2) / jnp.sqrt(var2 + 1e-5) * cst[5:6, :] + cst[6:7, :]

    c = jnp.maximum(
        jnp.dot(h2, wc1_ref[...].T, preferred_element_type=jnp.float32)
        + cst[8:9, 0:64], 0.0)
    out_ref[...] = jnp.dot(c, wc2_ref[...].T,
                           preferred_element_type=jnp.float32) + cst[7:8, :]


def _epilogue(a_t, g, dinv, wr, wh, m, wc1, wc2p, cst, BN):
    _, N, D = g.shape
    grid = (N // BN,)
    return pl.pallas_call(
        _epilogue_body,
        grid=grid,
        in_specs=[
            pl.BlockSpec((2, BN, D), lambda i: (0, i, 0)),
            pl.BlockSpec((2, BN, D), lambda i: (0, i, 0)),
            pl.BlockSpec((BN, 8), lambda i: (i, 0)),
            pl.BlockSpec((D, D), lambda i: (0, 0)),
            pl.BlockSpec((D, D), lambda i: (0, 0)),
            pl.BlockSpec((D, D), lambda i: (0, 0)),
            pl.BlockSpec((64, D), lambda i: (0, 0)),
            pl.BlockSpec((D, 64), lambda i: (0, 0)),
            pl.BlockSpec((16, D), lambda i: (0, 0)),
        ],
        out_specs=pl.BlockSpec((BN, D), lambda i: (i, 0)),
        out_shape=jax.ShapeDtypeStruct((N, D), jnp.float32),
    )(a_t, g, dinv, wr, wh, m, wc1, wc2p, cst)


def kernel(x, edge_index, timestamps, params):
    p = params
    N, F = x.shape
    E = timestamps.shape[0]
    D = p['W_in'].shape[0]

    info = plsc.get_sparse_core_info()
    NC, NS = info.num_cores, info.num_subcores
    RPW = ((N + NS - 1) // NS + 15) // 16 * 16
    RPW = ((RPW + 79) // 80) * 80
    NP = RPW * NS
    DUMP = ((N + 7) // 8) * 8 + 8

    src = edge_index[0]
    dst = edge_index[1]

    tbits = jax.lax.bitcast_convert_type(timestamps, jnp.int32)
    medbits = _median_bits(tbits.reshape(E // 128, 128), (E - 1) // 2 + 1)
    med_f = jax.lax.bitcast_convert_type(medbits[0, 0], jnp.float32)
    med8 = jnp.full((16,), med_f, jnp.float32)

    parts8 = _make_sc1(E, NP, DUMP, NC, NS)(dst, timestamps, med8)
    parts_t = parts8.T[:N]

    cstB = jnp.concatenate([
        p['b_in'][None, :], p['W_time'][:, 0][None, :], p['b_time'][None, :],
        jnp.zeros((5, D), jnp.float32)], axis=0)
    g, dinv = _prescale(x, parts_t, p['W_in'], cstB, 400)

    a_full = _make_sc2(E, N, D, NP, DUMP, NC, NS)(
        src, dst, timestamps, med8, g.reshape(2 * N, D))
    a_t = a_full[:, :N, :]

    wv = p['in_proj_W'][2 * D:3 * D]
    bv = p['in_proj_b'][2 * D:3 * D]
    m = p['out_proj_W'] @ wv
    c2 = bv @ p['out_proj_W'].T + p['out_proj_b']
    wc2p = jnp.zeros((D, 64), jnp.float32).at[0:2].set(p['Wc2'])
    bc2p = jnp.zeros((D,), jnp.float32).at[0:2].set(p['bc2'])
    bc1p = jnp.zeros((D,), jnp.float32).at[0:64].set(p['bc1'])
    cstC = jnp.concatenate([
        p['b_r'][None, :], p['b_h'][None, :], p['ln1_w'][None, :],
        p['ln1_b'][None, :], c2[None, :], p['ln2_w'][None, :],
        p['ln2_b'][None, :], bc2p[None, :], bc1p[None, :],
        jnp.zeros((7, D), jnp.float32)], axis=0)
    out128 = _epilogue(a_t, g, dinv, p['W_r'], p['W_h'], m, p['Wc1'], wc2p,
                       cstC, 400)
    return out128[:, :2]


# speedup vs baseline: 22.4835x; 1.2569x over previous
import functools

import jax
import jax.numpy as jnp
from jax import lax
from jax.experimental import pallas as pl
from jax.experimental.pallas import tpu as pltpu
from jax.experimental.pallas import tpu_sc as plsc


def _median_body(tb_ref, out_ref, *, kp1):
    tb = tb_ref[...]

    def body(_, lohi):
        lo, hi = lohi
        mid = (lo + hi) // 2
        cnt = jnp.sum((tb <= mid).astype(jnp.int32))
        ok = cnt >= kp1
        return jnp.where(ok, lo, mid + 1), jnp.where(ok, mid, hi)

    _, hi = lax.fori_loop(0, 31, body, (jnp.int32(0), jnp.int32(2**31 - 1)))
    out_ref[...] = jnp.full((8, 128), hi, jnp.int32)


def _median_bits(tbits2d, kp1):
    return pl.pallas_call(
        functools.partial(_median_body, kp1=kp1),
        out_shape=jax.ShapeDtypeStruct((8, 128), jnp.int32),
    )(tbits2d)


def _make_sc1(E, NP, DUMP, NC, NS):
    EW = E // (NC * NS)
    NB = EW // 80
    RPW = NP // NS
    mesh = plsc.VectorSubcoreMesh(core_axis_name="c", subcore_axis_name="s")

    @functools.partial(
        pl.kernel,
        out_type=jax.ShapeDtypeStruct((8, NP), jnp.float32),
        mesh=mesh,
        scratch_types=[
            pltpu.VMEM((EW,), jnp.int32),
            pltpu.VMEM((EW,), jnp.float32),
            pltpu.VMEM((NP,), jnp.float32),
            pltpu.VMEM((80,), jnp.int32),
            pltpu.VMEM((80,), jnp.int32),
            pltpu.VMEM((80,), jnp.float32),
            pltpu.VMEM((RPW,), jnp.float32),
            pltpu.VMEM((RPW,), jnp.float32),
            pltpu.VMEM((16,), jnp.float32),
            pltpu.VMEM_SHARED((NS, NP), jnp.float32),
            pltpu.VMEM_SHARED((NP,), jnp.float32),
            pltpu.VMEM_SHARED((NP,), jnp.float32),
        ],
    )
    def sc1(dst_hbm, ts_hbm, med_hbm, out_hbm, dstv, tsv, ntb, idxr, idxh,
            ones80, red, acc, medv, snt, sdegr, sdegh):
        c = lax.axis_index("c")
        s = lax.axis_index("s")
        gid = c * NS + s

        pltpu.sync_copy(med_hbm, medv)
        mvec = medv[...]

        def init_nt(i, _):
            ntb[pl.ds(i * 16, 16)] = jnp.full((16,), -jnp.inf, jnp.float32)
            return 0
        lax.fori_loop(0, NP // 16, init_nt, 0)

        def init_acc(i, _):
            acc[pl.ds(i * 16, 16)] = jnp.zeros((16,), jnp.float32)
            return 0
        lax.fori_loop(0, RPW // 16, init_acc, 0)
        for j in range(5):
            ones80[pl.ds(j * 16, 16)] = jnp.ones((16,), jnp.float32)

        pltpu.sync_copy(acc, sdegr.at[pl.ds(s * RPW, RPW)])
        pltpu.sync_copy(acc, sdegh.at[pl.ds(s * RPW, RPW)])
        plsc.subcore_barrier()

        base = gid * EW
        pltpu.sync_copy(dst_hbm.at[pl.ds(base, EW)], dstv)
        pltpu.sync_copy(ts_hbm.at[pl.ds(base, EW)], tsv)

        i16 = lax.iota(jnp.int32, 16)

        def batch(b, _):
            for j in range(5):
                off = b * 80 + j * 16
                d = dstv[pl.ds(off, 16)]
                t = tsv[pl.ds(off, 16)]
                rec = t >= mvec
                idxr[pl.ds(j * 16, 16)] = jnp.where(rec, d, DUMP)
                idxh[pl.ds(j * 16, 16)] = jnp.where(rec, DUMP, d)

                for k in range(16):
                    dk = d[k]
                    tk = t[k]
                    boff = (dk // 16) * 16
                    lanei = dk - boff
                    cur = ntb[pl.ds(boff, 16)]
                    ntb[pl.ds(boff, 16)] = jnp.where(
                        i16 == lanei, jnp.maximum(cur, tk), cur)
            pltpu.sync_copy(ones80, sdegr.at[idxr], add=True)
            pltpu.sync_copy(ones80, sdegh.at[idxh], add=True)
            return 0

        lax.fori_loop(0, NB, batch, 0)

        pltpu.sync_copy(ntb, snt.at[s])
        plsc.subcore_barrier()

        def init_r(i, _):
            acc[pl.ds(i * 16, 16)] = jnp.full((16,), -jnp.inf, jnp.float32)
            return 0
        lax.fori_loop(0, RPW // 16, init_r, 0)

        def red_k(k2, _):
            pltpu.sync_copy(snt.at[k2, pl.ds(s * RPW, RPW)], red)

            def mx(i, _):
                acc[pl.ds(i * 16, 16)] = jnp.maximum(
                    acc[pl.ds(i * 16, 16)], red[pl.ds(i * 16, 16)])
                return 0

            lax.fori_loop(0, RPW // 16, mx, 0)
            return 0

        lax.fori_loop(0, NS, red_k, 0)

        pltpu.sync_copy(acc, out_hbm.at[c, pl.ds(s * RPW, RPW)])
        pltpu.sync_copy(sdegr.at[pl.ds(s * RPW, RPW)],
                        out_hbm.at[2 + c, pl.ds(s * RPW, RPW)])
        pltpu.sync_copy(sdegh.at[pl.ds(s * RPW, RPW)],
                        out_hbm.at[4 + c, pl.ds(s * RPW, RPW)])

    return sc1


def _make_sc2(E, N, D, NP, DUMP, NC, NS):
    EW = E // NS
    NB = EW // 80
    RPW = NP // NS
    mesh = plsc.VectorSubcoreMesh(core_axis_name="c", subcore_axis_name="s")

    @functools.partial(
        pl.kernel,
        out_type=jax.ShapeDtypeStruct((2, NP, D), jnp.float32),
        mesh=mesh,
        scratch_types=[
            pltpu.VMEM((2000,), jnp.int32),
            pltpu.VMEM((2000,), jnp.int32),
            pltpu.VMEM((2000,), jnp.float32),
            pltpu.VMEM((80,), jnp.int32),
            pltpu.VMEM((80,), jnp.int32),
            pltpu.VMEM((80,), jnp.int32),
            pltpu.VMEM((80,), jnp.int32),
            pltpu.VMEM((80, D), jnp.float32),
            pltpu.VMEM((80, D), jnp.float32),
            pltpu.VMEM((16,), jnp.float32),
            pltpu.VMEM_SHARED((NP, D), jnp.float32),
            pltpu.SemaphoreType.DMA,
            pltpu.SemaphoreType.DMA,
            pltpu.SemaphoreType.DMA,
            pltpu.SemaphoreType.DMA,
        ],
    )
    def sc2(src_hbm, dst_hbm, ts_hbm, med_hbm, g_hbm, out_hbm, srcv, dstv,
            tsv, sidx0, sidx1, gidx0, gidx1, rows0, rows1, medv, acc,
            gsem0, gsem1, ssem0, ssem1):
        sidx = (sidx0, sidx1)
        gidx = (gidx0, gidx1)
        rows = (rows0, rows1)
        gsem = (gsem0, gsem1)
        ssem = (ssem0, ssem1)
        c = lax.axis_index("c")
        s = lax.axis_index("s")

        pltpu.sync_copy(med_hbm, medv)
        mvec = medv[...]

        def zr(i, _):
            def zc(k, _):
                rows0[i, pl.ds(k * 16, 16)] = jnp.zeros((16,), jnp.float32)
                return 0
            lax.fori_loop(0, D // 16, zc, 0)
            return 0
        lax.fori_loop(0, 80, zr, 0)

        def za(r, _):
            pltpu.sync_copy(rows0, acc.at[pl.ds(s * RPW + r * 80, 80)])
            return 0
        lax.fori_loop(0, RPW // 80, za, 0)
        plsc.subcore_barrier()

        base = s * EW
        goff = c * N

        def superb(sb, _):
            sbase = base + sb * 2000
            pltpu.sync_copy(src_hbm.at[pl.ds(sbase, 2000)], srcv)
            pltpu.sync_copy(dst_hbm.at[pl.ds(sbase, 2000)], dstv)
            pltpu.sync_copy(ts_hbm.at[pl.ds(sbase, 2000)], tsv)

            gd = [None, None]
            sd = [None, None]
            for b in range(25):
                p = b & 1
                if sd[p] is not None:
                    sd[p].wait()
                for j in range(5):
                    off = b * 80 + j * 16
                    d = dstv[pl.ds(off, 16)]
                    t = tsv[pl.ds(off, 16)]
                    reci = jnp.where(t >= mvec, 1, 0)
                    mine = reci == (1 - c)
                    sidx[p][pl.ds(j * 16, 16)] = jnp.where(mine, d, DUMP)
                    gidx[p][pl.ds(j * 16, 16)] = srcv[pl.ds(off, 16)] + goff
                gd[p] = pltpu.async_copy(g_hbm.at[gidx[p]], rows[p], gsem[p])
                if b >= 1:
                    q = 1 - p
                    gd[q].wait()
                    sd[q] = pltpu.async_copy(rows[q], acc.at[sidx[q]],
                                             ssem[q], add=True)
            gd[0].wait()
            sd[0] = pltpu.async_copy(rows0, acc.at[sidx0], ssem0, add=True)
            sd[0].wait()
            sd[1].wait()
            return 0

        lax.fori_loop(0, EW // 2000, superb, 0)
        plsc.subcore_barrier()

        def wout(r, _):
            pltpu.sync_copy(acc.at[pl.ds(s * RPW + r * 80, 80)],
                            out_hbm.at[c, pl.ds(s * RPW + r * 80, 80)])
            return 0

        lax.fori_loop(0, RPW // 80, wout, 0)

    return sc2


def _prescale_body(x_ref, pb_ref, pf_ref, win_ref, cst_ref, g_ref, dinv_ref):
    pf = pf_ref[...]
    ntc_f = jnp.maximum(pf[:, 0:1], pf[:, 1:2])
    ntf_f = jnp.where(ntc_f < -3e38, 0.0, ntc_f)
    mn = jnp.min(ntf_f)
    mx = jnp.max(ntf_f)

    pb = pb_ref[...]
    ntc = jnp.maximum(pb[:, 0:1], pb[:, 1:2])
    ntf = jnp.where(ntc < -3e38, 0.0, ntc)
    ntn = (ntf - mn) / (mx - mn + 1e-8)

    cst = cst_ref[...]
    h = jnp.dot(x_ref[...], win_ref[...].T,
                preferred_element_type=jnp.float32)
    h = jnp.maximum(h + cst[0:1, :] + ntn * cst[1:2, :] + cst[2:3, :], 0.0)
    dr = lax.rsqrt(pb[:, 2:3] + pb[:, 3:4] + 1.0)
    dh = lax.rsqrt(pb[:, 4:5] + pb[:, 5:6] + 1.0)
    g_ref[...] = jnp.stack([dr * h, dh * h])
    z = jnp.zeros_like(pb[:, 0:6])
    dinv_ref[...] = jnp.concatenate([dr, dh, z], axis=1)


def _prescale(x, parts_t, win, cst, BN):
    N, F = x.shape
    D = win.shape[0]
    grid = (N // BN,)
    return pl.pallas_call(
        _prescale_body,
        grid=grid,
        in_specs=[
            pl.BlockSpec((BN, F), lambda i: (i, 0)),
            pl.BlockSpec((BN, 8), lambda i: (i, 0)),
            pl.BlockSpec((N, 8), lambda i: (0, 0)),
            pl.BlockSpec((D, F), lambda i: (0, 0)),
            pl.BlockSpec((8, D), lambda i: (0, 0)),
        ],
        out_specs=[
            pl.BlockSpec((2, BN, D), lambda i: (0, i, 0)),
            pl.BlockSpec((BN, 8), lambda i: (i, 0)),
        ],
        out_shape=[
            jax.ShapeDtypeStruct((2, N, D), jnp.float32),
            jax.ShapeDtypeStruct((N, 8), jnp.float32),
        ],
    )(x, parts_t, parts_t, win, cst)


def _epilogue_body(a_ref, g_ref, dinv_ref, wr_ref, wh_ref, m_ref, wc1_ref,
                   wc2_ref, cst_ref, out_ref):
    cst = cst_ref[...]
    dv = dinv_ref[...]
    ar = (a_ref[0] + g_ref[0]) * dv[:, 0:1]
    ah = (a_ref[1] + g_ref[1]) * dv[:, 1:2]
    h_rec = jnp.dot(ar, wr_ref[...].T,
                    preferred_element_type=jnp.float32) + cst[0:1, :]
    h_his = jnp.dot(ah, wh_ref[...].T,
                    preferred_element_type=jnp.float32) + cst[1:2, :]
    m_r = jnp.mean(h_rec, axis=1, keepdims=True)
    m_h = jnp.mean(h_his, axis=1, keepdims=True)
    mxw = jnp.maximum(m_r, m_h)
    er = jnp.exp(m_r - mxw)
    eh = jnp.exp(m_h - mxw)
    den = er + eh
    summ = (er / den) * h_rec + (eh / den) * h_his

    mu = jnp.mean(summ, axis=1, keepdims=True)
    var = jnp.mean((summ - mu) ** 2, axis=1, keepdims=True)
    agg = (summ - mu) / jnp.sqrt(var + 1e-5) * cst[2:3, :] + cst[3:4, :]

    att = jnp.dot(agg, m_ref[...].T,
                  preferred_element_type=jnp.float32) + cst[4:5, :]
    z = agg + att
    mu2 = jnp.mean(z, axis=1, keepdims=True)
    var2 = jnp.mean((z - mu2) ** 2, axis=1, keepdims=True)
    h2 = (z - mu2) / jnp.sqrt(var2 + 1e-5) * cst[5:6, :] + cst[6:7, :]

    c = jnp.maximum(
        jnp.dot(h2, wc1_ref[...].T, preferred_element_type=jnp.float32)
        + cst[8:9, 0:64], 0.0)
    out_ref[...] = jnp.dot(c, wc2_ref[...].T,
                           preferred_element_type=jnp.float32) + cst[7:8, :]


def _epilogue(a_t, g, dinv, wr, wh, m, wc1, wc2p, cst, BN):
    _, N, D = g.shape
    grid = (N // BN,)
    return pl.pallas_call(
        _epilogue_body,
        grid=grid,
        in_specs=[
            pl.BlockSpec((2, BN, D), lambda i: (0, i, 0)),
            pl.BlockSpec((2, BN, D), lambda i: (0, i, 0)),
            pl.BlockSpec((BN, 8), lambda i: (i, 0)),
            pl.BlockSpec((D, D), lambda i: (0, 0)),
            pl.BlockSpec((D, D), lambda i: (0, 0)),
            pl.BlockSpec((D, D), lambda i: (0, 0)),
            pl.BlockSpec((64, D), lambda i: (0, 0)),
            pl.BlockSpec((D, 64), lambda i: (0, 0)),
            pl.BlockSpec((16, D), lambda i: (0, 0)),
        ],
        out_specs=pl.BlockSpec((BN, D), lambda i: (i, 0)),
        out_shape=jax.ShapeDtypeStruct((N, D), jnp.float32),
    )(a_t, g, dinv, wr, wh, m, wc1, wc2p, cst)


def kernel(x, edge_index, timestamps, params):
    p = params
    N, F = x.shape
    E = timestamps.shape[0]
    D = p['W_in'].shape[0]

    info = plsc.get_sparse_core_info()
    NC, NS = info.num_cores, info.num_subcores
    RPW = ((N + NS - 1) // NS + 15) // 16 * 16
    RPW = ((RPW + 79) // 80) * 80
    NP = RPW * NS
    DUMP = ((N + 7) // 8) * 8 + 8

    src = edge_index[0]
    dst = edge_index[1]

    tbits = jax.lax.bitcast_convert_type(timestamps, jnp.int32)
    medbits = _median_bits(tbits.reshape(E // 128, 128), (E - 1) // 2 + 1)
    med_f = jax.lax.bitcast_convert_type(medbits[0, 0], jnp.float32)
    med8 = jnp.full((16,), med_f, jnp.float32)

    parts8 = _make_sc1(E, NP, DUMP, NC, NS)(dst, timestamps, med8)
    parts_t = parts8.T[:N]

    cstB = jnp.concatenate([
        p['b_in'][None, :], p['W_time'][:, 0][None, :], p['b_time'][None, :],
        jnp.zeros((5, D), jnp.float32)], axis=0)
    g, dinv = _prescale(x, parts_t, p['W_in'], cstB, 400)

    a_full = _make_sc2(E, N, D, NP, DUMP, NC, NS)(
        src, dst, timestamps, med8, g.reshape(2 * N, D))
    a_t = a_full[:, :N, :]

    wv = p['in_proj_W'][2 * D:3 * D]
    bv = p['in_proj_b'][2 * D:3 * D]
    m = p['out_proj_W'] @ wv
    c2 = bv @ p['out_proj_W'].T + p['out_proj_b']
    wc2p = jnp.zeros((D, 64), jnp.float32).at[0:2].set(p['Wc2'])
    bc2p = jnp.zeros((D,), jnp.float32).at[0:2].set(p['bc2'])
    bc1p = jnp.zeros((D,), jnp.float32).at[0:64].set(p['bc1'])
    cstC = jnp.concatenate([
        p['b_r'][None, :], p['b_h'][None, :], p['ln1_w'][None, :],
        p['ln1_b'][None, :], c2[None, :], p['ln2_w'][None, :],
        p['ln2_b'][None, :], bc2p[None, :], bc1p[None, :],
        jnp.zeros((7, D), jnp.float32)], axis=0)
    out128 = _epilogue(a_t, g, dinv, p['W_r'], p['W_h'], m, p['Wc1'], wc2p,
                       cstC, 400)
    return out128[:, :2]
